# baseline (speedup 1.0000x reference)
import functools

import jax
import jax.numpy as jnp
from jax import lax
from jax.experimental import pallas as pl
from jax.experimental.pallas import tpu as pltpu
from jax.experimental.pallas import tpu_sc as plsc

B = 1024
L = 512
E = 128
VOCAB = 100000

NC = 2
NS = 16
NW = NC * NS
BPW = B // NW
CL = 128
NLC = L // CL
LANES = 16
EV = E // LANES
NBUF = 4


def _emb_body(seq_hbm, seg_hbm, tok_hbm, pos_hbm, segtab_hbm, out_hbm,
              idx_v, seg_v, rows_v, pos_v, segtab_v,
              gsem, osem, psem):
    cid = lax.axis_index("c")
    sid = lax.axis_index("s")
    wid = sid * NC + cid
    wbase = wid * BPW * L

    pltpu.sync_copy(segtab_hbm, segtab_v)
    r0 = [segtab_v[0, pl.ds(j * LANES, LANES)] for j in range(EV)]
    d1 = [segtab_v[1, pl.ds(j * LANES, LANES)] - r0[j] for j in range(EV)]
    d2 = [segtab_v[2, pl.ds(j * LANES, LANES)] - r0[j] for j in range(EV)]

    def issue(c, lcbase):
        p = lax.rem(c, NBUF)
        base = pl.multiple_of(lcbase + c * L, CL)
        pltpu.sync_copy(seq_hbm.at[pl.ds(base, CL)], idx_v.at[p])
        pltpu.sync_copy(seg_hbm.at[pl.ds(base, CL)], seg_v.at[p])
        pltpu.async_copy(tok_hbm.at[idx_v.at[p]],
                         rows_v.at[pl.ds(p * CL, CL)], gsem.at[p])

    def wait_write(c, lcbase):
        p = lax.rem(c, NBUF)
        base = pl.multiple_of(lcbase + c * L, CL)
        pltpu.make_async_copy(rows_v.at[pl.ds(p * CL, CL)],
                              out_hbm.at[pl.ds(base, CL)], osem.at[p]).wait()

    def step(c, lcbase, pbase):
        p = lax.rem(c, NBUF)
        roff = p * CL
        base = pl.multiple_of(lcbase + c * L, CL)
        pltpu.make_async_copy(tok_hbm.at[idx_v.at[p]],
                              rows_v.at[pl.ds(roff, CL)], gsem.at[p]).wait()

        def group_body(g, _):
            i0 = pl.multiple_of(g * LANES, LANES)
            segf = seg_v[p, pl.ds(i0, LANES)].astype(jnp.float32)
            m1v = segf * (2.0 - segf)
            m2v = segf * (segf - 1.0) * 0.5
            for k in range(LANES):
                m1 = jnp.broadcast_to(m1v[k], (LANES,))
                m2 = jnp.broadcast_to(m2v[k], (LANES,))
                r = roff + i0 + k
                pr = pbase + i0 + k
                for j in range(EV):
                    sl = pl.ds(j * LANES, LANES)
                    rows_v[r, sl] = (rows_v[r, sl] + pos_v[pr, sl]
                                     + d1[j] * m1 + d2[j] * m2)
            return 0

        lax.fori_loop(0, CL // LANES, group_body, 0)
        pltpu.async_copy(rows_v.at[pl.ds(roff, CL)],
                         out_hbm.at[pl.ds(base, CL)], osem.at[p])

    pltpu.async_copy(pos_hbm.at[pl.ds(0, CL)], pos_v.at[pl.ds(0, CL)], psem)

    def lc_body(lc, _):
        lcbase = wbase + lc * CL
        pbase = pl.multiple_of(lax.rem(lc, 2) * CL, CL)
        pltpu.make_async_copy(pos_hbm.at[pl.ds(0, CL)],
                              pos_v.at[pl.ds(pbase, CL)], psem).wait()

        def pos_body(i, _):
            for j in range(EV):
                sl = pl.ds(j * LANES, LANES)
                pos_v[pbase + i, sl] = pos_v[pbase + i, sl] + r0[j]
            return 0

        lax.fori_loop(0, CL, pos_body, 0)

        issue(0, lcbase)
        issue(1, lcbase)

        def chunk_body(c, _):
            @pl.when(c >= 2)
            def _():
                wait_write(c - 2, lcbase)

            @pl.when(c + 2 < BPW)
            def _():
                issue(c + 2, lcbase)

            step(c, lcbase, pbase)
            return 0

        lax.fori_loop(0, BPW, chunk_body, 0)

        @pl.when(lc + 1 < NLC)
        def _():
            nb = pl.multiple_of(lax.rem(lc + 1, 2) * CL, CL)
            pltpu.async_copy(pos_hbm.at[pl.ds((lc + 1) * CL, CL)],
                             pos_v.at[pl.ds(nb, CL)], psem)

        wait_write(BPW - 2, lcbase)
        wait_write(BPW - 1, lcbase)
        return 0

    lax.fori_loop(0, NLC, lc_body, 0)


@functools.partial(jax.jit, static_argnames=())
def kernel(sequence, segment_label, token_table, position_table,
           segment_table):
    seq = sequence.reshape(-1).astype(jnp.int32)
    seg = segment_label.reshape(-1).astype(jnp.int32)

    mesh = plsc.VectorSubcoreMesh(core_axis_name="c", subcore_axis_name="s",
                                  num_cores=NC, num_subcores=NS)
    out = pl.kernel(
        _emb_body,
        out_type=jax.ShapeDtypeStruct((B * L, E), jnp.float32),
        mesh=mesh,
        scratch_types=[
            pltpu.VMEM((NBUF, CL), jnp.int32),
            pltpu.VMEM((NBUF, CL), jnp.int32),
            pltpu.VMEM((NBUF * CL, E), jnp.float32),
            pltpu.VMEM((2 * CL, E), jnp.float32),
            pltpu.VMEM((3, E), jnp.float32),
            pltpu.SemaphoreType.DMA((NBUF,)),
            pltpu.SemaphoreType.DMA((NBUF,)),
            pltpu.SemaphoreType.DMA,
        ],
    )(seq, seg, token_table, position_table, segment_table)
    return out.reshape(B, L, E)
